# baseline (speedup 1.0000x reference)
import jax
import jax.numpy as jnp
from jax import lax
from jax.experimental import pallas as pl
from jax.experimental.pallas import tpu as pltpu
from jax.experimental.pallas import tpu_sc as plsc


def _sc_body(pose_hbm, out_hbm, out_v):
    wid = lax.axis_index("s") * 2 + lax.axis_index("c")
    base = wid * 512
    pltpu.sync_copy(out_v, out_hbm.at[pl.ds(base, 512)])


def kernel(pose):
    mesh = plsc.VectorSubcoreMesh(core_axis_name="c", subcore_axis_name="s")
    k = pl.kernel(
        _sc_body,
        mesh=mesh,
        out_type=jax.ShapeDtypeStruct((16384, 4), jnp.float32),
        scratch_types=[pltpu.VMEM((512, 4), jnp.float32)],
        compiler_params=pltpu.CompilerParams(needs_layout_passes=False),
    )
    return k(pose)


# speedup vs baseline: 1.0499x; 1.0499x over previous
import jax
import jax.numpy as jnp
from jax.experimental import pallas as pl


def _onehot(d):
    k = jax.lax.broadcasted_iota(jnp.int32, (d, 4), 0)
    j = jax.lax.broadcasted_iota(jnp.int32, (d, 4), 1)
    hit = lambda kk, jj: ((k == kk) & (j == jj)).astype(jnp.float32)
    return hit(52, 0) - hit(55, 1) - hit(9, 2) - hit(12, 3)


def _angle_prior_kernel(x_ref, out_ref):
    g = jnp.dot(x_ref[...], _onehot(x_ref.shape[1]),
                preferred_element_type=jnp.float32,
                precision=jax.lax.Precision.HIGHEST)
    e = jnp.exp(g)
    out_ref[...] = e * e


def kernel(pose):
    n, d = pose.shape
    block = 1024
    return pl.pallas_call(
        _angle_prior_kernel,
        grid=(n // block,),
        in_specs=[pl.BlockSpec((block, d), lambda i: (i, 0))],
        out_specs=pl.BlockSpec((block, 4), lambda i: (i, 0)),
        out_shape=jax.ShapeDtypeStruct((n, 4), pose.dtype),
    )(pose)


# speedup vs baseline: 1.3465x; 1.2825x over previous
import jax
import jax.numpy as jnp
from jax.experimental import pallas as pl

_BLOCK = 2048
_HALF_BLOCKS = 4


def _onehot(d):
    k = jax.lax.broadcasted_iota(jnp.int32, (d, 4), 0)
    j = jax.lax.broadcasted_iota(jnp.int32, (d, 4), 1)
    hit = lambda kk, jj: ((k == kk) & (j == jj)).astype(jnp.float32)
    return hit(52, 0) - hit(55, 1) - hit(9, 2) - hit(12, 3)


def _angle_prior_kernel(xa_ref, xb_ref, out_ref):
    s = _onehot(xa_ref.shape[1])
    for h, x_ref in ((0, xa_ref), (1, xb_ref)):
        g = jnp.dot(x_ref[...], s,
                    preferred_element_type=jnp.float32,
                    precision=jax.lax.Precision.HIGHEST)
        e = jnp.exp(g)
        out_ref[h] = e * e


def kernel(pose):
    n, d = pose.shape
    out = pl.pallas_call(
        _angle_prior_kernel,
        grid=(_HALF_BLOCKS,),
        in_specs=[
            pl.BlockSpec((_BLOCK, d), lambda i: (i, 0)),
            pl.BlockSpec((_BLOCK, d), lambda i: (i + _HALF_BLOCKS, 0)),
        ],
        out_specs=pl.BlockSpec((2, _BLOCK, 4), lambda i: (0, i, 0)),
        out_shape=jax.ShapeDtypeStruct((2, n // 2, 4), pose.dtype),
    )(pose, pose)
    return out.reshape(n, 4)
